# baseline (speedup 1.0000x reference)
import functools

import jax
import jax.numpy as jnp
from jax import lax
from jax.experimental import pallas as pl
from jax.experimental.pallas import tpu as pltpu
from jax.experimental.pallas import tpu_sc as plsc

H = 25000
HP = 25088
NP = 2 * HP
D = 64
TILES = 16
EC = 128
NSS = 396
UNROLL = 12
NI = NSS // UNROLL
EPT = NSS * EC
NCH = HP // EC
RT = 13
DEG_UNROLL = 6
DEG_NI = NSS // DEG_UNROLL

_MESH = dict(
    mesh=plsc.VectorSubcoreMesh(core_axis_name="c", subcore_axis_name="s"),
    compiler_params=pltpu.CompilerParams(use_tc_tiling_on_sc=False),
)


def _when(cond, fn):
    if cond is None:
        fn()
    else:
        pl.when(cond)(fn)


@functools.partial(
    pl.kernel,
    out_type=jax.ShapeDtypeStruct((NP, 16), jnp.float32),
    scratch_types=(
        [pltpu.VMEM((2, EC), jnp.int32) for _ in range(6)]
        + [
            pltpu.VMEM((EC, 16), jnp.float32),
            pltpu.VMEM((EC, 16), jnp.float32),
            pltpu.VMEM_SHARED((HP, 16), jnp.float32),
        ]
        + [pltpu.SemaphoreType.DMA for _ in range(8)]
    ),
    **_MESH,
)
def _deg_kernel(cidx_hbm, deg_hbm, i0, i1, i2, i3, i4, i5, ones_v, stage_v,
                deg_sh, s0, s1, s2, s3, s4, s5, ssem, zsem):
    c = lax.axis_index("c")
    s = lax.axis_index("s")
    ib = (i0, i1, i2, i3, i4, i5)
    isems = (s0, s1, s2, s3, s4, s5)
    blk0 = (c * TILES + s) * NSS

    def fill(i, carry):
        ones_v[i, pl.ds(0, 16)] = jnp.full((16,), 1.0, jnp.float32)
        stage_v[i, pl.ds(0, 16)] = jnp.zeros((16,), jnp.float32)
        return carry

    lax.fori_loop(0, EC, fill, 0)

    def zero_chunk(t, carry):
        j = s + TILES * t

        @pl.when(j < NCH)
        def _():
            pltpu.async_copy(stage_v, deg_sh.at[pl.ds(j * EC, EC)], zsem)

        return carry

    lax.fori_loop(0, RT, zero_chunk, 0)

    def zero_wait(t, carry):
        j = s + TILES * t

        @pl.when(j < NCH)
        def _():
            pltpu.make_async_copy(stage_v, deg_sh.at[pl.ds(j * EC, EC)], zsem).wait()

        return carry

    lax.fori_loop(0, RT, zero_wait, 0)
    plsc.subcore_barrier()

    for q in range(4):
        pltpu.async_copy(cidx_hbm.at[blk0 + q], ib[q], isems[q])

    def sstep(ss, Q, c_drain, c_pref):
        def _a():
            pltpu.make_async_copy(ones_v, deg_sh.at[pl.ds(0, EC)], ssem).wait()

        _when(c_drain, _a)
        pltpu.make_async_copy(cidx_hbm.at[blk0 + ss], ib[Q], isems[Q]).wait()
        pltpu.async_copy(ones_v, deg_sh.at[ib[Q].at[1]], ssem, add=True)

        def _d():
            pq = (Q + 4) % DEG_UNROLL
            pltpu.async_copy(cidx_hbm.at[blk0 + ss + 4], ib[pq], isems[pq])

        _when(c_pref, _d)

    def body(i, carry):
        ss0 = i * DEG_UNROLL
        for p in range(DEG_UNROLL):
            c_drain = (i > 0) if p < 2 else None
            c_pref = (i < DEG_NI - 1) if p >= 2 else None
            sstep(ss0 + p, p, c_drain, c_pref)
        return carry

    lax.fori_loop(0, DEG_NI, body, 0)
    for _ in range(2):
        pltpu.make_async_copy(ones_v, deg_sh.at[pl.ds(0, EC)], ssem).wait()
    plsc.subcore_barrier()

    def writeback(t, carry):
        j = s + TILES * t

        @pl.when(j < NCH)
        def _():
            pltpu.sync_copy(deg_sh.at[pl.ds(j * EC, EC)], stage_v)
            pltpu.sync_copy(stage_v, deg_hbm.at[pl.ds(c * HP + j * EC, EC)])

        return carry

    lax.fori_loop(0, RT, writeback, 0)


@functools.partial(
    pl.kernel,
    out_type=(
        jax.ShapeDtypeStruct((NP, D), jnp.float32),
        jax.ShapeDtypeStruct((NP, D), jnp.float32),
    ),
    scratch_types=(
        [pltpu.VMEM((2, EC), jnp.int32) for _ in range(4)]
        + [pltpu.VMEM((EC, D), jnp.float32) for _ in range(3)]
        + [
            pltpu.VMEM((EC,), jnp.float32),
            pltpu.VMEM_SHARED((HP, D), jnp.float32),
        ]
        + [pltpu.SemaphoreType.DMA for _ in range(9)]
    ),
    **_MESH,
)
def _layer_kernel(cidx_hbm, y_hbm, x_hbm, dis_hbm, xo_hbm, yo_hbm,
                  i0, i1, i2, i3, r0, r1, r2, dis_v, s_sh,
                  s0, s1, s2, s3, g0, g1, g2, ssem, zsem):
    c = lax.axis_index("c")
    s = lax.axis_index("s")
    ib = (i0, i1, i2, i3)
    rb = (r0, r1, r2)
    isems = (s0, s1, s2, s3)
    gsems = (g0, g1, g2)
    blk0 = (c * TILES + s) * NSS

    def zfill(i, carry):
        for q in range(4):
            r0[i, pl.ds(q * 16, 16)] = jnp.zeros((16,), jnp.float32)
        return carry

    lax.fori_loop(0, EC, zfill, 0)

    def zero_chunk(t, carry):
        j = s + TILES * t

        @pl.when(j < NCH)
        def _():
            pltpu.async_copy(r0, s_sh.at[pl.ds(j * EC, EC)], zsem)

        return carry

    lax.fori_loop(0, RT, zero_chunk, 0)

    def zero_wait(t, carry):
        j = s + TILES * t

        @pl.when(j < NCH)
        def _():
            pltpu.make_async_copy(r0, s_sh.at[pl.ds(j * EC, EC)], zsem).wait()

        return carry

    lax.fori_loop(0, RT, zero_wait, 0)
    plsc.subcore_barrier()

    for q in range(3):
        pltpu.async_copy(cidx_hbm.at[blk0 + q], ib[q], isems[q])
    for q in range(2):
        pltpu.make_async_copy(cidx_hbm.at[blk0 + q], ib[q], isems[q]).wait()
        pltpu.async_copy(y_hbm.at[ib[q].at[0]], rb[q], gsems[q])

    def sstep(ss, P, Q, c_drain, c_next, c_pref):
        pltpu.make_async_copy(y_hbm.at[pl.ds(0, EC)], rb[P], gsems[P]).wait()

        def _b():
            pltpu.make_async_copy(rb[(P + 2) % 3], s_sh.at[pl.ds(0, EC)], ssem).wait()

        _when(c_drain, _b)

        def _c():
            nq = (Q + 2) % 4
            np_ = (P + 2) % 3
            pltpu.make_async_copy(cidx_hbm.at[blk0 + ss + 2], ib[nq], isems[nq]).wait()
            pltpu.async_copy(y_hbm.at[ib[nq].at[0]], rb[np_], gsems[np_])

        _when(c_next, _c)

        pltpu.async_copy(rb[P], s_sh.at[ib[Q].at[1]], ssem, add=True)

        def _e():
            pq = (Q + 3) % 4
            pltpu.async_copy(cidx_hbm.at[blk0 + ss + 3], ib[pq], isems[pq])

        _when(c_pref, _e)

    def body(i, carry):
        ss0 = i * UNROLL
        nlast = i < NI - 1
        for p in range(UNROLL):
            c_drain = (i > 0) if p == 0 else None
            c_next = nlast if p >= UNROLL - 2 else None
            c_pref = nlast if p >= UNROLL - 3 else None
            sstep(ss0 + p, p % 3, p % 4, c_drain, c_next, c_pref)
        return carry

    lax.fori_loop(0, NI, body, 0)
    pltpu.make_async_copy(rb[(NSS - 1) % 3], s_sh.at[pl.ds(0, EC)], ssem).wait()
    plsc.subcore_barrier()

    def rescale(t, carry):
        j = s + TILES * t

        @pl.when(j < NCH)
        def _():
            row0 = c * HP + j * EC
            pltpu.sync_copy(s_sh.at[pl.ds(j * EC, EC)], r0)
            pltpu.sync_copy(x_hbm.at[pl.ds(row0, EC)], r1)
            pltpu.sync_copy(dis_hbm.at[pl.ds(row0, EC)], dis_v)

            def rowf(gi, carry2):
                dvec = dis_v[pl.ds(gi * 16, 16)]
                d2vec = dvec * dvec
                for k in range(16):
                    r = gi * 16 + k
                    dsc = dvec[k]
                    d2 = d2vec[k]
                    for q in range(4):
                        sl = pl.ds(q * 16, 16)
                        sv = r0[r, sl]
                        r1[r, sl] = r1[r, sl] + dsc * sv
                        r0[r, sl] = d2 * sv
                return carry2

            lax.fori_loop(0, EC // 16, rowf, 0)
            pltpu.sync_copy(r1, xo_hbm.at[pl.ds(row0, EC)])
            pltpu.sync_copy(r0, yo_hbm.at[pl.ds(row0, EC)])

        return carry

    lax.fori_loop(0, RT, rescale, 0)


def kernel(edge_index, user_weight, item_weight):
    src = edge_index[0].astype(jnp.int32)
    dst = edge_index[1].astype(jnp.int32)
    n_edges = src.shape[0]
    total = TILES * EPT
    pad = total - n_edges

    order = jnp.argsort((dst >= H).astype(jnp.int32), stable=True)
    src = src[order]
    dst = dst[order]
    src_p = jnp.where(src >= H, src + (HP - H), src)
    m0 = dst < H
    mi0 = jnp.where(m0, 1, 0)
    cs0 = jnp.cumsum(mi0)
    n0 = cs0[-1]
    cs1 = jnp.cumsum(1 - mi0)
    n0r = ((n0 + 127) // 128) * 128
    pos = jnp.where(m0, cs0 - 1, n0r + cs1 - 1)
    _epadn = 6528 * 128
    gsrc = jnp.zeros((_epadn,), jnp.int32).at[pos].set(src_p, unique_indices=True)
    gdst = jnp.full((_epadn,), -1, jnp.int32).at[pos].set(dst, unique_indices=True)
    src_all = jnp.concatenate([src_p, jnp.zeros((pad,), jnp.int32)])
    dst_all = jnp.concatenate([dst, jnp.full((pad,), -1, jnp.int32)])
    trash = H + (jnp.arange(total, dtype=jnp.int32) & 63)
    dst0 = jnp.where((dst_all >= 0) & (dst_all < H), dst_all, trash)
    dst1 = jnp.where(dst_all >= H, dst_all - H, trash)

    srcr = src_all.reshape(TILES, NSS, 1, EC)
    blk0_ = jnp.concatenate([srcr, dst0.reshape(TILES, NSS, 1, EC)], axis=2)
    blk1_ = jnp.concatenate([srcr, dst1.reshape(TILES, NSS, 1, EC)], axis=2)
    cidx = jnp.stack([blk0_, blk1_]).reshape(2 * TILES * NSS, 2, EC)

    deg16 = _deg_kernel(cidx)
    deg = deg16[:, 0]
    rows = jnp.arange(NP)
    real = (rows % HP) < H
    dis = jnp.where(real & (deg > 0), lax.rsqrt(jnp.maximum(deg, 1.0)), 0.0)
    dis = dis.astype(jnp.float32)

    x = jnp.concatenate([
        jnp.pad(user_weight, ((0, HP - H), (0, 0))),
        jnp.pad(item_weight, ((0, HP - H), (0, 0))),
    ])
    x = x + 0.0 * (gsrc[0] + gdst[0]).astype(jnp.float32)
    y = dis[:, None] * x
    for _ in range(3):
        x, y = _layer_kernel(cidx, y, x, dis)
    return (x[:H], x[HP:HP + H])


# speedup vs baseline: 2.7863x; 2.7863x over previous
import functools

import jax
import jax.numpy as jnp
from jax import lax
from jax.experimental import pallas as pl
from jax.experimental.pallas import tpu as pltpu
from jax.experimental.pallas import tpu_sc as plsc

H = 25000
HP = 25088
NP = 2 * HP
D = 64
TILES = 16
EC = 128
NSS = 396
UNROLL = 12
NI = NSS // UNROLL
EPT = NSS * EC
NCH = HP // EC
RT = 13
DEG_UNROLL = 6
DEG_NI = NSS // DEG_UNROLL

_MESH = dict(
    mesh=plsc.VectorSubcoreMesh(core_axis_name="c", subcore_axis_name="s"),
    compiler_params=pltpu.CompilerParams(use_tc_tiling_on_sc=False),
)


def _when(cond, fn):
    if cond is None:
        fn()
    else:
        pl.when(cond)(fn)


@functools.partial(
    pl.kernel,
    out_type=jax.ShapeDtypeStruct((NP, 16), jnp.float32),
    scratch_types=(
        [pltpu.VMEM((2, EC), jnp.int32) for _ in range(6)]
        + [
            pltpu.VMEM((EC, 16), jnp.float32),
            pltpu.VMEM((EC, 16), jnp.float32),
            pltpu.VMEM_SHARED((HP, 16), jnp.float32),
        ]
        + [pltpu.SemaphoreType.DMA for _ in range(8)]
    ),
    **_MESH,
)
def _deg_kernel(cidx_hbm, deg_hbm, i0, i1, i2, i3, i4, i5, ones_v, stage_v,
                deg_sh, s0, s1, s2, s3, s4, s5, ssem, zsem):
    c = lax.axis_index("c")
    s = lax.axis_index("s")
    ib = (i0, i1, i2, i3, i4, i5)
    isems = (s0, s1, s2, s3, s4, s5)
    blk0 = (c * TILES + s) * NSS

    def fill(i, carry):
        ones_v[i, pl.ds(0, 16)] = jnp.full((16,), 1.0, jnp.float32)
        stage_v[i, pl.ds(0, 16)] = jnp.zeros((16,), jnp.float32)
        return carry

    lax.fori_loop(0, EC, fill, 0)

    def zero_chunk(t, carry):
        j = s + TILES * t

        @pl.when(j < NCH)
        def _():
            pltpu.async_copy(stage_v, deg_sh.at[pl.ds(j * EC, EC)], zsem)

        return carry

    lax.fori_loop(0, RT, zero_chunk, 0)

    def zero_wait(t, carry):
        j = s + TILES * t

        @pl.when(j < NCH)
        def _():
            pltpu.make_async_copy(stage_v, deg_sh.at[pl.ds(j * EC, EC)], zsem).wait()

        return carry

    lax.fori_loop(0, RT, zero_wait, 0)
    plsc.subcore_barrier()

    for q in range(4):
        pltpu.async_copy(cidx_hbm.at[blk0 + q], ib[q], isems[q])

    def sstep(ss, Q, c_drain, c_pref):
        def _a():
            pltpu.make_async_copy(ones_v, deg_sh.at[pl.ds(0, EC)], ssem).wait()

        _when(c_drain, _a)
        pltpu.make_async_copy(cidx_hbm.at[blk0 + ss], ib[Q], isems[Q]).wait()
        pltpu.async_copy(ones_v, deg_sh.at[ib[Q].at[1]], ssem, add=True)

        def _d():
            pq = (Q + 4) % DEG_UNROLL
            pltpu.async_copy(cidx_hbm.at[blk0 + ss + 4], ib[pq], isems[pq])

        _when(c_pref, _d)

    def body(i, carry):
        ss0 = i * DEG_UNROLL
        for p in range(DEG_UNROLL):
            c_drain = (i > 0) if p < 2 else None
            c_pref = (i < DEG_NI - 1) if p >= 2 else None
            sstep(ss0 + p, p, c_drain, c_pref)
        return carry

    lax.fori_loop(0, DEG_NI, body, 0)
    for _ in range(2):
        pltpu.make_async_copy(ones_v, deg_sh.at[pl.ds(0, EC)], ssem).wait()
    plsc.subcore_barrier()

    def writeback(t, carry):
        j = s + TILES * t

        @pl.when(j < NCH)
        def _():
            pltpu.sync_copy(deg_sh.at[pl.ds(j * EC, EC)], stage_v)
            pltpu.sync_copy(stage_v, deg_hbm.at[pl.ds(c * HP + j * EC, EC)])

        return carry

    lax.fori_loop(0, RT, writeback, 0)


@functools.partial(
    pl.kernel,
    out_type=(
        jax.ShapeDtypeStruct((NP, D), jnp.float32),
        jax.ShapeDtypeStruct((NP, D), jnp.float32),
    ),
    scratch_types=(
        [pltpu.VMEM((2, EC), jnp.int32) for _ in range(4)]
        + [pltpu.VMEM((EC, D), jnp.float32) for _ in range(3)]
        + [
            pltpu.VMEM((EC,), jnp.float32),
            pltpu.VMEM_SHARED((HP, D), jnp.float32),
        ]
        + [pltpu.SemaphoreType.DMA for _ in range(9)]
    ),
    **_MESH,
)
def _layer_kernel(cidx_hbm, y_hbm, x_hbm, dis_hbm, xo_hbm, yo_hbm,
                  i0, i1, i2, i3, r0, r1, r2, dis_v, s_sh,
                  s0, s1, s2, s3, g0, g1, g2, ssem, zsem):
    c = lax.axis_index("c")
    s = lax.axis_index("s")
    ib = (i0, i1, i2, i3)
    rb = (r0, r1, r2)
    isems = (s0, s1, s2, s3)
    gsems = (g0, g1, g2)
    blk0 = (c * TILES + s) * NSS

    def zfill(i, carry):
        for q in range(4):
            r0[i, pl.ds(q * 16, 16)] = jnp.zeros((16,), jnp.float32)
        return carry

    lax.fori_loop(0, EC, zfill, 0)

    def zero_chunk(t, carry):
        j = s + TILES * t

        @pl.when(j < NCH)
        def _():
            pltpu.async_copy(r0, s_sh.at[pl.ds(j * EC, EC)], zsem)

        return carry

    lax.fori_loop(0, RT, zero_chunk, 0)

    def zero_wait(t, carry):
        j = s + TILES * t

        @pl.when(j < NCH)
        def _():
            pltpu.make_async_copy(r0, s_sh.at[pl.ds(j * EC, EC)], zsem).wait()

        return carry

    lax.fori_loop(0, RT, zero_wait, 0)
    plsc.subcore_barrier()

    for q in range(3):
        pltpu.async_copy(cidx_hbm.at[blk0 + q], ib[q], isems[q])
    for q in range(2):
        pltpu.make_async_copy(cidx_hbm.at[blk0 + q], ib[q], isems[q]).wait()
        pltpu.async_copy(y_hbm.at[ib[q].at[0]], rb[q], gsems[q])

    def sstep(ss, P, Q, c_drain, c_next, c_pref):
        pltpu.make_async_copy(y_hbm.at[pl.ds(0, EC)], rb[P], gsems[P]).wait()

        def _b():
            pltpu.make_async_copy(rb[(P + 2) % 3], s_sh.at[pl.ds(0, EC)], ssem).wait()

        _when(c_drain, _b)

        def _c():
            nq = (Q + 2) % 4
            np_ = (P + 2) % 3
            pltpu.make_async_copy(cidx_hbm.at[blk0 + ss + 2], ib[nq], isems[nq]).wait()
            pltpu.async_copy(y_hbm.at[ib[nq].at[0]], rb[np_], gsems[np_])

        _when(c_next, _c)

        pltpu.async_copy(rb[P], s_sh.at[ib[Q].at[1]], ssem, add=True)

        def _e():
            pq = (Q + 3) % 4
            pltpu.async_copy(cidx_hbm.at[blk0 + ss + 3], ib[pq], isems[pq])

        _when(c_pref, _e)

    def body(i, carry):
        ss0 = i * UNROLL
        nlast = i < NI - 1
        for p in range(UNROLL):
            c_drain = (i > 0) if p == 0 else None
            c_next = nlast if p >= UNROLL - 2 else None
            c_pref = nlast if p >= UNROLL - 3 else None
            sstep(ss0 + p, p % 3, p % 4, c_drain, c_next, c_pref)
        return carry

    lax.fori_loop(0, NI, body, 0)
    pltpu.make_async_copy(rb[(NSS - 1) % 3], s_sh.at[pl.ds(0, EC)], ssem).wait()
    plsc.subcore_barrier()

    def rescale(t, carry):
        j = s + TILES * t

        @pl.when(j < NCH)
        def _():
            row0 = c * HP + j * EC
            pltpu.sync_copy(s_sh.at[pl.ds(j * EC, EC)], r0)
            pltpu.sync_copy(x_hbm.at[pl.ds(row0, EC)], r1)
            pltpu.sync_copy(dis_hbm.at[pl.ds(row0, EC)], dis_v)

            def rowf(gi, carry2):
                dvec = dis_v[pl.ds(gi * 16, 16)]
                d2vec = dvec * dvec
                for k in range(16):
                    r = gi * 16 + k
                    dsc = dvec[k]
                    d2 = d2vec[k]
                    for q in range(4):
                        sl = pl.ds(q * 16, 16)
                        sv = r0[r, sl]
                        r1[r, sl] = r1[r, sl] + dsc * sv
                        r0[r, sl] = d2 * sv
                return carry2

            lax.fori_loop(0, EC // 16, rowf, 0)
            pltpu.sync_copy(r1, xo_hbm.at[pl.ds(row0, EC)])
            pltpu.sync_copy(r0, yo_hbm.at[pl.ds(row0, EC)])

        return carry

    lax.fori_loop(0, RT, rescale, 0)


def kernel(edge_index, user_weight, item_weight):
    src = edge_index[0].astype(jnp.int32)
    dst = edge_index[1].astype(jnp.int32)
    n_edges = src.shape[0]
    total = TILES * EPT
    pad = total - n_edges

    order = jnp.argsort((dst >= H).astype(jnp.int32), stable=True)
    src = src[order]
    dst = dst[order]
    src_p = jnp.where(src >= H, src + (HP - H), src)
    src_all = jnp.concatenate([src_p, jnp.zeros((pad,), jnp.int32)])
    dst_all = jnp.concatenate([dst, jnp.full((pad,), -1, jnp.int32)])
    trash = H + (jnp.arange(total, dtype=jnp.int32) & 63)
    dst0 = jnp.where((dst_all >= 0) & (dst_all < H), dst_all, trash)
    dst1 = jnp.where(dst_all >= H, dst_all - H, trash)

    srcr = src_all.reshape(TILES, NSS, 1, EC)
    blk0_ = jnp.concatenate([srcr, dst0.reshape(TILES, NSS, 1, EC)], axis=2)
    blk1_ = jnp.concatenate([srcr, dst1.reshape(TILES, NSS, 1, EC)], axis=2)
    cidx = jnp.stack([blk0_, blk1_]).reshape(2 * TILES * NSS, 2, EC)

    deg16 = _deg_kernel(cidx)
    deg = deg16[:, 0]
    rows = jnp.arange(NP)
    real = (rows % HP) < H
    dis = jnp.where(real & (deg > 0), lax.rsqrt(jnp.maximum(deg, 1.0)), 0.0)
    dis = dis.astype(jnp.float32)

    x = jnp.concatenate([
        jnp.pad(user_weight, ((0, HP - H), (0, 0))),
        jnp.pad(item_weight, ((0, HP - H), (0, 0))),
    ])
    y = dis[:, None] * x
    for _ in range(3):
        x, y = _layer_kernel(cidx, y, x, dis)
    return (x[:H], x[HP:HP + H])


# speedup vs baseline: 3.9963x; 1.4342x over previous
import functools

import jax
import jax.numpy as jnp
from jax import lax
from jax.experimental import pallas as pl
from jax.experimental.pallas import tpu as pltpu
from jax.experimental.pallas import tpu_sc as plsc

H = 25000
HP = 25088
NP = 2 * HP
D = 64
TILES = 16
EC = 128
NSS = 396
UNROLL = 12
NI = NSS // UNROLL
EPT = NSS * EC
NCH = HP // EC
RT = 13
DEG_UNROLL = 6
DEG_NI = NSS // DEG_UNROLL

_MESH = dict(
    mesh=plsc.VectorSubcoreMesh(core_axis_name="c", subcore_axis_name="s"),
    compiler_params=pltpu.CompilerParams(use_tc_tiling_on_sc=False),
)


def _when(cond, fn):
    if cond is None:
        fn()
    else:
        pl.when(cond)(fn)


@functools.partial(
    pl.kernel,
    out_type=jax.ShapeDtypeStruct((NP, 16), jnp.float32),
    scratch_types=(
        [pltpu.VMEM((2, EC), jnp.int32) for _ in range(6)]
        + [
            pltpu.VMEM((EC, 16), jnp.float32),
            pltpu.VMEM((EC, 16), jnp.float32),
            pltpu.VMEM_SHARED((HP, 16), jnp.float32),
        ]
        + [pltpu.SemaphoreType.DMA for _ in range(8)]
    ),
    **_MESH,
)
def _deg_kernel(cidx_hbm, deg_hbm, i0, i1, i2, i3, i4, i5, ones_v, stage_v,
                deg_sh, s0, s1, s2, s3, s4, s5, ssem, zsem):
    c = lax.axis_index("c")
    s = lax.axis_index("s")
    ib = (i0, i1, i2, i3, i4, i5)
    isems = (s0, s1, s2, s3, s4, s5)
    blk0 = (c * TILES + s) * NSS

    def fill(i, carry):
        ones_v[i, pl.ds(0, 16)] = jnp.full((16,), 1.0, jnp.float32)
        stage_v[i, pl.ds(0, 16)] = jnp.zeros((16,), jnp.float32)
        return carry

    lax.fori_loop(0, EC, fill, 0)

    def zero_chunk(t, carry):
        j = s + TILES * t

        @pl.when(j < NCH)
        def _():
            pltpu.async_copy(stage_v, deg_sh.at[pl.ds(j * EC, EC)], zsem)

        return carry

    lax.fori_loop(0, RT, zero_chunk, 0)

    def zero_wait(t, carry):
        j = s + TILES * t

        @pl.when(j < NCH)
        def _():
            pltpu.make_async_copy(stage_v, deg_sh.at[pl.ds(j * EC, EC)], zsem).wait()

        return carry

    lax.fori_loop(0, RT, zero_wait, 0)
    plsc.subcore_barrier()

    for q in range(4):
        pltpu.async_copy(cidx_hbm.at[blk0 + q], ib[q], isems[q])

    def sstep(ss, Q, c_drain, c_pref):
        def _a():
            pltpu.make_async_copy(ones_v, deg_sh.at[pl.ds(0, EC)], ssem).wait()

        _when(c_drain, _a)
        pltpu.make_async_copy(cidx_hbm.at[blk0 + ss], ib[Q], isems[Q]).wait()
        pltpu.async_copy(ones_v, deg_sh.at[ib[Q].at[1]], ssem, add=True)

        def _d():
            pq = (Q + 4) % DEG_UNROLL
            pltpu.async_copy(cidx_hbm.at[blk0 + ss + 4], ib[pq], isems[pq])

        _when(c_pref, _d)

    def body(i, carry):
        ss0 = i * DEG_UNROLL
        for p in range(DEG_UNROLL):
            c_drain = (i > 0) if p < 2 else None
            c_pref = (i < DEG_NI - 1) if p >= 2 else None
            sstep(ss0 + p, p, c_drain, c_pref)
        return carry

    lax.fori_loop(0, DEG_NI, body, 0)
    for _ in range(2):
        pltpu.make_async_copy(ones_v, deg_sh.at[pl.ds(0, EC)], ssem).wait()
    plsc.subcore_barrier()

    def writeback(t, carry):
        j = s + TILES * t

        @pl.when(j < NCH)
        def _():
            pltpu.sync_copy(deg_sh.at[pl.ds(j * EC, EC)], stage_v)
            pltpu.sync_copy(stage_v, deg_hbm.at[pl.ds(c * HP + j * EC, EC)])

        return carry

    lax.fori_loop(0, RT, writeback, 0)


@functools.partial(
    pl.kernel,
    out_type=(
        jax.ShapeDtypeStruct((NP, D), jnp.float32),
        jax.ShapeDtypeStruct((NP, D), jnp.float32),
    ),
    scratch_types=(
        [pltpu.VMEM((2, EC), jnp.int32) for _ in range(4)]
        + [pltpu.VMEM((EC, D), jnp.float32) for _ in range(3)]
        + [
            pltpu.VMEM((EC,), jnp.float32),
            pltpu.VMEM_SHARED((HP, D), jnp.float32),
        ]
        + [pltpu.SemaphoreType.DMA for _ in range(9)]
    ),
    **_MESH,
)
def _layer_kernel(cidx_hbm, y_hbm, x_hbm, dis_hbm, xo_hbm, yo_hbm,
                  i0, i1, i2, i3, r0, r1, r2, dis_v, s_sh,
                  s0, s1, s2, s3, g0, g1, g2, ssem, zsem):
    c = lax.axis_index("c")
    s = lax.axis_index("s")
    ib = (i0, i1, i2, i3)
    rb = (r0, r1, r2)
    isems = (s0, s1, s2, s3)
    gsems = (g0, g1, g2)
    blk0 = (c * TILES + s) * NSS

    def zfill(i, carry):
        for q in range(4):
            r0[i, pl.ds(q * 16, 16)] = jnp.zeros((16,), jnp.float32)
        return carry

    lax.fori_loop(0, EC, zfill, 0)

    def zero_chunk(t, carry):
        j = s + TILES * t

        @pl.when(j < NCH)
        def _():
            pltpu.async_copy(r0, s_sh.at[pl.ds(j * EC, EC)], zsem)

        return carry

    lax.fori_loop(0, RT, zero_chunk, 0)

    def zero_wait(t, carry):
        j = s + TILES * t

        @pl.when(j < NCH)
        def _():
            pltpu.make_async_copy(r0, s_sh.at[pl.ds(j * EC, EC)], zsem).wait()

        return carry

    lax.fori_loop(0, RT, zero_wait, 0)
    plsc.subcore_barrier()

    for q in range(3):
        pltpu.async_copy(cidx_hbm.at[blk0 + q], ib[q], isems[q])
    for q in range(2):
        pltpu.make_async_copy(cidx_hbm.at[blk0 + q], ib[q], isems[q]).wait()
        pltpu.async_copy(y_hbm.at[ib[q].at[0]], rb[q], gsems[q])

    def sstep(ss, P, Q, c_drain, c_next, c_pref):
        pltpu.make_async_copy(y_hbm.at[pl.ds(0, EC)], rb[P], gsems[P]).wait()

        def _b():
            pltpu.make_async_copy(rb[(P + 2) % 3], s_sh.at[pl.ds(0, EC)], ssem).wait()

        _when(c_drain, _b)

        def _c():
            nq = (Q + 2) % 4
            np_ = (P + 2) % 3
            pltpu.make_async_copy(cidx_hbm.at[blk0 + ss + 2], ib[nq], isems[nq]).wait()
            pltpu.async_copy(y_hbm.at[ib[nq].at[0]], rb[np_], gsems[np_])

        _when(c_next, _c)

        pltpu.async_copy(rb[P], s_sh.at[ib[Q].at[1]], ssem, add=True)

        def _e():
            pq = (Q + 3) % 4
            pltpu.async_copy(cidx_hbm.at[blk0 + ss + 3], ib[pq], isems[pq])

        _when(c_pref, _e)

    def body(i, carry):
        ss0 = i * UNROLL
        nlast = i < NI - 1
        for p in range(UNROLL):
            c_drain = (i > 0) if p == 0 else None
            c_next = nlast if p >= UNROLL - 2 else None
            c_pref = nlast if p >= UNROLL - 3 else None
            sstep(ss0 + p, p % 3, p % 4, c_drain, c_next, c_pref)
        return carry

    lax.fori_loop(0, NI, body, 0)
    pltpu.make_async_copy(rb[(NSS - 1) % 3], s_sh.at[pl.ds(0, EC)], ssem).wait()
    plsc.subcore_barrier()

    def rescale(t, carry):
        j = s + TILES * t

        @pl.when(j < NCH)
        def _():
            row0 = c * HP + j * EC
            pltpu.sync_copy(s_sh.at[pl.ds(j * EC, EC)], r0)
            pltpu.sync_copy(x_hbm.at[pl.ds(row0, EC)], r1)
            pltpu.sync_copy(dis_hbm.at[pl.ds(row0, EC)], dis_v)

            def rowf(gi, carry2):
                dvec = dis_v[pl.ds(gi * 16, 16)]
                d2vec = dvec * dvec
                for k in range(16):
                    r = gi * 16 + k
                    dsc = dvec[k]
                    d2 = d2vec[k]
                    for q in range(4):
                        sl = pl.ds(q * 16, 16)
                        sv = r0[r, sl]
                        r1[r, sl] = r1[r, sl] + dsc * sv
                        r0[r, sl] = d2 * sv
                return carry2

            lax.fori_loop(0, EC // 16, rowf, 0)
            pltpu.sync_copy(r1, xo_hbm.at[pl.ds(row0, EC)])
            pltpu.sync_copy(r0, yo_hbm.at[pl.ds(row0, EC)])

        return carry

    lax.fori_loop(0, RT, rescale, 0)


def kernel(edge_index, user_weight, item_weight):
    src = edge_index[0].astype(jnp.int32)
    dst = edge_index[1].astype(jnp.int32)
    n_edges = src.shape[0]
    total = TILES * EPT
    pad = total - n_edges

    src_p = jnp.where(src >= H, src + (HP - H), src)
    src_all = jnp.concatenate([src_p, jnp.zeros((pad,), jnp.int32)])
    dst_all = jnp.concatenate([dst, jnp.full((pad,), -1, jnp.int32)])
    trash = H + (jnp.arange(total, dtype=jnp.int32) & 63)
    dst0 = jnp.where((dst_all >= 0) & (dst_all < H), dst_all, trash)
    dst1 = jnp.where(dst_all >= H, dst_all - H, trash)

    srcr = src_all.reshape(TILES, NSS, 1, EC)
    blk0_ = jnp.concatenate([srcr, dst0.reshape(TILES, NSS, 1, EC)], axis=2)
    blk1_ = jnp.concatenate([srcr, dst1.reshape(TILES, NSS, 1, EC)], axis=2)
    cidx = jnp.stack([blk0_, blk1_]).reshape(2 * TILES * NSS, 2, EC)

    deg16 = _deg_kernel(cidx)
    deg = deg16[:, 0]
    rows = jnp.arange(NP)
    real = (rows % HP) < H
    dis = jnp.where(real & (deg > 0), lax.rsqrt(jnp.maximum(deg, 1.0)), 0.0)
    dis = dis.astype(jnp.float32)

    x = jnp.concatenate([
        jnp.pad(user_weight, ((0, HP - H), (0, 0))),
        jnp.pad(item_weight, ((0, HP - H), (0, 0))),
    ])
    y = dis[:, None] * x
    for _ in range(3):
        x, y = _layer_kernel(cidx, y, x, dis)
    return (x[:H], x[HP:HP + H])
